# Initial kernel scaffold; baseline (speedup 1.0000x reference)
import jax
import jax.numpy as jnp
from jax.experimental import pallas as pl


def kernel(input_ids, emb_table, pos_table, W1, b1, W2, b2):
    raise NotImplementedError("write your pallas kernel here")


# speedup vs baseline: 5.4992x; 5.4992x over previous
import functools

import jax
import jax.numpy as jnp
from jax import lax
from jax.experimental import pallas as pl
from jax.experimental.pallas import tpu as pltpu
from jax.experimental.pallas import tpu_sc as plsc

_LANES = 16


def _sc_gather_sum(ids, table):
    B, L = ids.shape
    _, H = table.shape
    NC, NS = 2, 16
    NW = NC * NS
    RPW = B // NW
    CH = H // _LANES

    mesh = plsc.VectorSubcoreMesh(core_axis_name="c", subcore_axis_name="s")

    @functools.partial(
        pl.kernel,
        out_type=jax.ShapeDtypeStruct((B, H), jnp.float32),
        mesh=mesh,
        scratch_types=[
            pltpu.VMEM((RPW, L), jnp.int32),
            pltpu.VMEM((L, H), jnp.float32),
            pltpu.VMEM((L, H), jnp.float32),
            pltpu.VMEM((RPW, H), jnp.float32),
            pltpu.SemaphoreType.DMA,
            pltpu.SemaphoreType.DMA,
        ],
    )
    def sc_kernel(ids_hbm, table_hbm, out_hbm, idx_v, buf0, buf1, out_v, sem0, sem1):
        wid = lax.axis_index("s") * NC + lax.axis_index("c")
        base = wid * RPW
        pltpu.sync_copy(ids_hbm.at[pl.ds(base, RPW)], idx_v)

        def reduce_into(buf, r):
            def body(i, accs):
                return tuple(accs[j] + buf[i, pl.ds(_LANES * j, _LANES)]
                             for j in range(CH))
            accs = lax.fori_loop(
                0, L, body,
                tuple(jnp.zeros((_LANES,), jnp.float32) for _ in range(CH)))
            for j in range(CH):
                out_v[r, pl.ds(_LANES * j, _LANES)] = accs[j]

        pltpu.async_copy(table_hbm.at[idx_v.at[0]], buf0, sem0)

        def loop_body(g, carry):
            r0 = 2 * g
            pltpu.async_copy(table_hbm.at[idx_v.at[r0 + 1]], buf1, sem1)
            pltpu.make_async_copy(table_hbm.at[idx_v.at[r0]], buf0, sem0).wait()
            reduce_into(buf0, r0)

            @pl.when(r0 + 2 < RPW)
            def _():
                pltpu.async_copy(table_hbm.at[idx_v.at[r0 + 2]], buf0, sem0)

            pltpu.make_async_copy(table_hbm.at[idx_v.at[r0 + 1]], buf1, sem1).wait()
            reduce_into(buf1, r0 + 1)
            return carry

        lax.fori_loop(0, RPW // 2, loop_body, 0)
        pltpu.sync_copy(out_v, out_hbm.at[pl.ds(base, RPW)])

    return sc_kernel(ids, table)


def _tc_mlp(sums, pos, W1, b1, W2, b2, inv_l):
    B, H = sums.shape
    L = pos.shape[0]
    BM = 512

    def mlp_kernel(s_ref, pos_ref, w1_ref, b1_ref, w2_ref, b2_ref, o_ref):
        pos_mean = jnp.mean(pos_ref[...], axis=0, keepdims=True)
        x = s_ref[...] * inv_l + pos_mean
        h = lax.dot_general(x, w1_ref[...], (((1,), (1,)), ((), ())),
                            preferred_element_type=jnp.float32)
        h = jnp.maximum(h + b1_ref[...], 0.0)
        o = lax.dot_general(h, w2_ref[...], (((1,), (1,)), ((), ())),
                            preferred_element_type=jnp.float32)
        o_ref[...] = o + b2_ref[...]

    return pl.pallas_call(
        mlp_kernel,
        grid=(B // BM,),
        in_specs=[
            pl.BlockSpec((BM, H), lambda i: (i, 0)),
            pl.BlockSpec((L, H), lambda i: (0, 0)),
            pl.BlockSpec(W1.shape, lambda i: (0, 0)),
            pl.BlockSpec((1, H), lambda i: (0, 0)),
            pl.BlockSpec(W2.shape, lambda i: (0, 0)),
            pl.BlockSpec((1, H), lambda i: (0, 0)),
        ],
        out_specs=pl.BlockSpec((BM, W2.shape[0]), lambda i: (i, 0)),
        out_shape=jax.ShapeDtypeStruct((B, W2.shape[0]), jnp.float32),
    )(sums, pos, W1, b1, W2, b2)


def kernel(input_ids, emb_table, pos_table, W1, b1, W2, b2):
    ids = input_ids.astype(jnp.int32)
    L = ids.shape[1]
    sums = _sc_gather_sum(ids, emb_table)
    return _tc_mlp(sums, pos_table[:L], W1, b1.reshape(1, -1),
                   W2, b2.reshape(1, -1), 1.0 / L)
